# baseline (speedup 1.0000x reference)
import jax
import jax.numpy as jnp
from jax import lax
from jax.experimental import pallas as pl
from jax.experimental.pallas import tpu as pltpu
from jax.experimental.pallas import tpu_sc as plsc

N = 10000
D = 128
E = 320000

NC = 2
NS = 16
NW = NC * NS
EPT = E // NW
CHUNK = 125
CPT = 80

N_SP = 10240
N_DEG = 10240


def _ln_body(x_ref, g_ref, b_ref, o_ref):
    x = x_ref[...]
    mu = jnp.mean(x, axis=1, keepdims=True)
    xc = x - mu
    var = jnp.mean(xc * xc, axis=1, keepdims=True)
    o_ref[...] = xc * lax.rsqrt(var + 1e-5) * g_ref[...] + b_ref[...]


def _layernorm_tc(x, g, b):
    blk = 1000
    return pl.pallas_call(
        _ln_body,
        grid=(N // blk,),
        in_specs=[
            pl.BlockSpec((blk, D), lambda i: (i, 0)),
            pl.BlockSpec((1, D), lambda i: (0, 0)),
            pl.BlockSpec((1, D), lambda i: (0, 0)),
        ],
        out_specs=pl.BlockSpec((blk, D), lambda i: (i, 0)),
        out_shape=jax.ShapeDtypeStruct((N, D), jnp.float32),
    )(x, g.reshape(1, D), b.reshape(1, D))


def _edge_kernel(hn, er, zeros2d, zeros1d,
                 agg_out, deg_out,
                 src_v, dst_v, rowbuf0, rowbuf1, ones_v, agg_sp, deg_sp,
                 gsem0, gsem1, ssem0, ssem1, dsem):
    cid = lax.axis_index("c")
    sid = lax.axis_index("s")
    wid = cid * NS + sid

    pltpu.sync_copy(zeros2d, agg_sp.at[pl.ds(sid * 640, 640)])
    pltpu.sync_copy(zeros1d.at[pl.ds(sid * 640, 640)],
                    deg_sp.at[pl.ds(sid * 640, 640)])
    for i in range(8):
        ones_v[pl.ds(i * 16, 16)] = jnp.full((16,), 1.0, jnp.float32)
    plsc.subcore_barrier()

    half = CPT // 2
    npairs = half // 2
    ones_c = ones_v.at[pl.ds(0, CHUNK)]

    def gath(j, buf, sem):
        return pltpu.make_async_copy(hn.at[src_v.at[j]], buf, sem)

    def scat(j, buf, sem):
        return pltpu.make_async_copy(buf, agg_sp.at[dst_v.at[j]], sem)

    def body(g, carry):
        a = 2 * g
        gath(a, rowbuf0, gsem0).wait()
        pltpu.async_copy(rowbuf0, agg_sp.at[dst_v.at[a]], ssem0, add=True)
        pltpu.async_copy(ones_c, deg_sp.at[dst_v.at[a]], dsem, add=True)
        gath(a + 1, rowbuf1, gsem1).wait()
        pltpu.async_copy(rowbuf1, agg_sp.at[dst_v.at[a + 1]], ssem1, add=True)
        pltpu.async_copy(ones_c, deg_sp.at[dst_v.at[a + 1]], dsem, add=True)

        @pl.when(g < npairs - 1)
        def _():
            scat(a, rowbuf0, ssem0).wait()
            pltpu.async_copy(hn.at[src_v.at[a + 2]], rowbuf0, gsem0)
            scat(a + 1, rowbuf1, ssem1).wait()
            pltpu.async_copy(hn.at[src_v.at[a + 3]], rowbuf1, gsem1)

        pltpu.make_async_copy(ones_c, deg_sp.at[dst_v.at[a]], dsem).wait()
        pltpu.make_async_copy(ones_c, deg_sp.at[dst_v.at[a + 1]], dsem).wait()
        return carry

    last = 2 * (npairs - 1)
    for h in range(2):
        pltpu.sync_copy(er.at[wid * 2 + h], src_v)
        pltpu.sync_copy(er.at[NW * 2 + wid * 2 + h], dst_v)
        pltpu.async_copy(hn.at[src_v.at[0]], rowbuf0, gsem0)
        pltpu.async_copy(hn.at[src_v.at[1]], rowbuf1, gsem1)
        lax.fori_loop(0, npairs, body, 0)
        scat(last, rowbuf0, ssem0).wait()
        scat(last + 1, rowbuf1, ssem1).wait()
    plsc.subcore_barrier()

    pltpu.sync_copy(agg_sp.at[pl.ds(sid * 640, 640)],
                    agg_out.at[cid, pl.ds(sid * 640, 640)])
    pltpu.sync_copy(deg_sp.at[pl.ds(sid * 640, 640)],
                    deg_out.at[cid, pl.ds(sid * 640, 640)])


def _edge_aggregate_sc(hn, er, zeros2d, zeros1d):
    mesh = plsc.VectorSubcoreMesh(core_axis_name="c", subcore_axis_name="s")
    return pl.kernel(
        _edge_kernel,
        mesh=mesh,
        out_type=[
            jax.ShapeDtypeStruct((NC, N_SP, D), jnp.float32),
            jax.ShapeDtypeStruct((NC, N_DEG), jnp.float32),
        ],
        scratch_types=[
            pltpu.VMEM((CPT // 2, CHUNK), jnp.int32),
            pltpu.VMEM((CPT // 2, CHUNK), jnp.int32),
            pltpu.VMEM((CHUNK, D), jnp.float32),
            pltpu.VMEM((CHUNK, D), jnp.float32),
            pltpu.VMEM((128,), jnp.float32),
            pltpu.VMEM_SHARED((N_SP, D), jnp.float32),
            pltpu.VMEM_SHARED((N_DEG,), jnp.float32),
            pltpu.SemaphoreType.DMA,
            pltpu.SemaphoreType.DMA,
            pltpu.SemaphoreType.DMA,
            pltpu.SemaphoreType.DMA,
            pltpu.SemaphoreType.DMA,
        ],
    )(hn, er, zeros2d, zeros1d)


def _tail_body(hn_ref, a0_ref, a1_ref, d0_ref, d1_ref,
               ws_ref, wn_ref, wsi_ref, b_ref, ing_ref, inb_ref, bsi_ref,
               o_ref):
    hn = hn_ref[...]
    agg = a0_ref[0] + a1_ref[0]
    deg = jnp.maximum(d0_ref[0] + d1_ref[0], 1.0)
    h_neigh = agg / deg
    dn = (((1,), (1,)), ((), ()))
    h_conv = (lax.dot_general(hn, ws_ref[...], dn,
                              preferred_element_type=jnp.float32)
              + lax.dot_general(h_neigh, wn_ref[...], dn,
                                preferred_element_type=jnp.float32)
              + b_ref[...])
    h1 = h_conv + hn
    mu = jnp.mean(h1, axis=1, keepdims=True)
    xc = h1 - mu
    var = jnp.mean(xc * xc, axis=1, keepdims=True)
    h2 = xc * lax.rsqrt(var + 1e-5) * ing_ref[...] + inb_ref[...]
    z = lax.dot_general(h2, wsi_ref[...], dn,
                        preferred_element_type=jnp.float32) + bsi_ref[...]
    h3 = jnp.where(z > 0, z, jnp.exp(jnp.minimum(z, 0.0)) - 1.0)
    o_ref[...] = h3 + h2


def _dense_tail_tc(hn, agg, deg,
                   W_self, W_neigh, W_si, b, in_g, in_b, b_si):
    blk = 1000
    row = lambda i: (i, 0)
    full = lambda i: (0, 0)
    return pl.pallas_call(
        _tail_body,
        grid=(N // blk,),
        in_specs=[
            pl.BlockSpec((blk, D), row),
            pl.BlockSpec((1, blk, D), lambda i: (0, i, 0)),
            pl.BlockSpec((1, blk, D), lambda i: (1, i, 0)),
            pl.BlockSpec((1, blk, 1), lambda i: (0, i, 0)),
            pl.BlockSpec((1, blk, 1), lambda i: (1, i, 0)),
            pl.BlockSpec((D, D), full),
            pl.BlockSpec((D, D), full),
            pl.BlockSpec((D, D), full),
            pl.BlockSpec((1, D), full),
            pl.BlockSpec((1, D), full),
            pl.BlockSpec((1, D), full),
            pl.BlockSpec((1, D), full),
        ],
        out_specs=pl.BlockSpec((blk, D), row),
        out_shape=jax.ShapeDtypeStruct((N, D), jnp.float32),
    )(hn, agg, agg, deg, deg, W_self, W_neigh, W_si,
      b.reshape(1, D), in_g.reshape(1, D), in_b.reshape(1, D),
      b_si.reshape(1, D))


def kernel(h, edge_index, W_self, W_neigh, b, ln_g, ln_b, in_g, in_b,
           W_si, b_si):
    hn = _layernorm_tc(h, ln_g, ln_b)

    er = edge_index.reshape(2 * NW * 2, CPT // 2, CHUNK)
    zeros2d = jnp.zeros((640, D), jnp.float32)
    zeros1d = jnp.zeros((N_DEG,), jnp.float32)

    agg, deg = _edge_aggregate_sc(hn, er, zeros2d, zeros1d)

    return _dense_tail_tc(hn, agg, deg.reshape(NC, N_DEG, 1),
                          W_self, W_neigh, W_si, b, in_g, in_b, b_si)


# speedup vs baseline: 1.1688x; 1.1688x over previous
import jax
import jax.numpy as jnp
from jax import lax
from jax.experimental import pallas as pl
from jax.experimental.pallas import tpu as pltpu
from jax.experimental.pallas import tpu_sc as plsc

N = 10000
D = 128
E = 320000

NC = 2
NS = 16
NW = NC * NS
EPT = E // NW
CHUNK = 100
CPT = 100
SEC = 25

N_SP = 10240
N_DEG = 10240


def _ln_body(x_ref, g_ref, b_ref, o_ref):
    x = x_ref[...]
    mu = jnp.mean(x, axis=1, keepdims=True)
    xc = x - mu
    var = jnp.mean(xc * xc, axis=1, keepdims=True)
    o_ref[...] = xc * lax.rsqrt(var + 1e-5) * g_ref[...] + b_ref[...]


def _layernorm_tc(x, g, b):
    blk = 1000
    return pl.pallas_call(
        _ln_body,
        grid=(N // blk,),
        in_specs=[
            pl.BlockSpec((blk, D), lambda i: (i, 0)),
            pl.BlockSpec((1, D), lambda i: (0, 0)),
            pl.BlockSpec((1, D), lambda i: (0, 0)),
        ],
        out_specs=pl.BlockSpec((blk, D), lambda i: (i, 0)),
        out_shape=jax.ShapeDtypeStruct((N, D), jnp.float32),
    )(x, g.reshape(1, D), b.reshape(1, D))


def _edge_kernel(hn, er, zeros2d, zeros1d,
                 agg_out, deg_out,
                 src_v, dst_v, rowbuf0, rowbuf1, rowbuf2, ones_v,
                 agg_sp, deg_sp, gsem0, gsem1, gsem2, dsem):
    cid = lax.axis_index("c")
    sid = lax.axis_index("s")
    wid = cid * NS + sid

    pltpu.sync_copy(zeros2d, agg_sp.at[pl.ds(sid * 640, 640)])
    pltpu.sync_copy(zeros1d.at[pl.ds(sid * 640, 640)],
                    deg_sp.at[pl.ds(sid * 640, 640)])
    for i in range(8):
        ones_v[pl.ds(i * 16, 16)] = jnp.full((16,), 1.0, jnp.float32)
    plsc.subcore_barrier()

    ntrip = (SEC - 1) // 3
    ones_c = ones_v.at[pl.ds(0, CHUNK)]

    def enq(j, buf, sem):
        pltpu.async_copy(hn.at[src_v.at[j]], buf, sem)

    def process(j, buf, sem):
        pltpu.make_async_copy(hn.at[src_v.at[j]], buf, sem).wait()
        pltpu.async_copy(ones_c, deg_sp.at[dst_v.at[j]], dsem, add=True)
        pltpu.sync_copy(buf, agg_sp.at[dst_v.at[j]], add=True)
        pltpu.make_async_copy(ones_c, deg_sp.at[dst_v.at[j]], dsem).wait()

    def body(g, carry):
        a = 3 * g
        process(a, rowbuf0, gsem0)
        enq(a + 3, rowbuf0, gsem0)
        process(a + 1, rowbuf1, gsem1)

        @pl.when(g < ntrip - 1)
        def _():
            enq(a + 4, rowbuf1, gsem1)

        process(a + 2, rowbuf2, gsem2)

        @pl.when(g < ntrip - 1)
        def _():
            enq(a + 5, rowbuf2, gsem2)

        return carry

    for s in range(4):
        pltpu.sync_copy(er.at[wid * 4 + s], src_v)
        pltpu.sync_copy(er.at[NW * 4 + wid * 4 + s], dst_v)
        enq(0, rowbuf0, gsem0)
        enq(1, rowbuf1, gsem1)
        enq(2, rowbuf2, gsem2)
        lax.fori_loop(0, ntrip, body, 0)
        process(SEC - 1, rowbuf0, gsem0)
    plsc.subcore_barrier()

    pltpu.sync_copy(agg_sp.at[pl.ds(sid * 640, 640)],
                    agg_out.at[cid, pl.ds(sid * 640, 640)])
    pltpu.sync_copy(deg_sp.at[pl.ds(sid * 640, 640)],
                    deg_out.at[cid, pl.ds(sid * 640, 640)])


def _edge_aggregate_sc(hn, er, zeros2d, zeros1d):
    mesh = plsc.VectorSubcoreMesh(core_axis_name="c", subcore_axis_name="s")
    return pl.kernel(
        _edge_kernel,
        mesh=mesh,
        out_type=[
            jax.ShapeDtypeStruct((NC, N_SP, D), jnp.float32),
            jax.ShapeDtypeStruct((NC, N_DEG), jnp.float32),
        ],
        scratch_types=[
            pltpu.VMEM((SEC, CHUNK), jnp.int32),
            pltpu.VMEM((SEC, CHUNK), jnp.int32),
            pltpu.VMEM((CHUNK, D), jnp.float32),
            pltpu.VMEM((CHUNK, D), jnp.float32),
            pltpu.VMEM((CHUNK, D), jnp.float32),
            pltpu.VMEM((128,), jnp.float32),
            pltpu.VMEM_SHARED((N_SP, D), jnp.float32),
            pltpu.VMEM_SHARED((N_DEG,), jnp.float32),
            pltpu.SemaphoreType.DMA,
            pltpu.SemaphoreType.DMA,
            pltpu.SemaphoreType.DMA,
            pltpu.SemaphoreType.DMA,
        ],
    )(hn, er, zeros2d, zeros1d)


def _tail_body(hn_ref, a0_ref, a1_ref, d0_ref, d1_ref,
               ws_ref, wn_ref, wsi_ref, b_ref, ing_ref, inb_ref, bsi_ref,
               o_ref):
    hn = hn_ref[...]
    agg = a0_ref[0] + a1_ref[0]
    deg = jnp.maximum(d0_ref[0] + d1_ref[0], 1.0)
    h_neigh = agg / deg
    dn = (((1,), (1,)), ((), ()))
    h_conv = (lax.dot_general(hn, ws_ref[...], dn,
                              preferred_element_type=jnp.float32)
              + lax.dot_general(h_neigh, wn_ref[...], dn,
                                preferred_element_type=jnp.float32)
              + b_ref[...])
    h1 = h_conv + hn
    mu = jnp.mean(h1, axis=1, keepdims=True)
    xc = h1 - mu
    var = jnp.mean(xc * xc, axis=1, keepdims=True)
    h2 = xc * lax.rsqrt(var + 1e-5) * ing_ref[...] + inb_ref[...]
    z = lax.dot_general(h2, wsi_ref[...], dn,
                        preferred_element_type=jnp.float32) + bsi_ref[...]
    h3 = jnp.where(z > 0, z, jnp.exp(jnp.minimum(z, 0.0)) - 1.0)
    o_ref[...] = h3 + h2


def _dense_tail_tc(hn, agg, deg,
                   W_self, W_neigh, W_si, b, in_g, in_b, b_si):
    blk = 1000
    row = lambda i: (i, 0)
    full = lambda i: (0, 0)
    return pl.pallas_call(
        _tail_body,
        grid=(N // blk,),
        in_specs=[
            pl.BlockSpec((blk, D), row),
            pl.BlockSpec((1, blk, D), lambda i: (0, i, 0)),
            pl.BlockSpec((1, blk, D), lambda i: (1, i, 0)),
            pl.BlockSpec((1, blk, 1), lambda i: (0, i, 0)),
            pl.BlockSpec((1, blk, 1), lambda i: (1, i, 0)),
            pl.BlockSpec((D, D), full),
            pl.BlockSpec((D, D), full),
            pl.BlockSpec((D, D), full),
            pl.BlockSpec((1, D), full),
            pl.BlockSpec((1, D), full),
            pl.BlockSpec((1, D), full),
            pl.BlockSpec((1, D), full),
        ],
        out_specs=pl.BlockSpec((blk, D), row),
        out_shape=jax.ShapeDtypeStruct((N, D), jnp.float32),
    )(hn, agg, agg, deg, deg, W_self, W_neigh, W_si,
      b.reshape(1, D), in_g.reshape(1, D), in_b.reshape(1, D),
      b_si.reshape(1, D))


def kernel(h, edge_index, W_self, W_neigh, b, ln_g, ln_b, in_g, in_b,
           W_si, b_si):
    hn = _layernorm_tc(h, ln_g, ln_b)

    er = edge_index.reshape(2 * NW * 4, SEC, CHUNK)
    zeros2d = jnp.zeros((640, D), jnp.float32)
    zeros1d = jnp.zeros((N_DEG,), jnp.float32)

    agg, deg = _edge_aggregate_sc(hn, er, zeros2d, zeros1d)

    return _dense_tail_tc(hn, agg, deg.reshape(NC, N_DEG, 1),
                          W_self, W_neigh, W_si, b, in_g, in_b, b_si)


# speedup vs baseline: 1.1795x; 1.0091x over previous
import jax
import jax.numpy as jnp
from jax import lax
from jax.experimental import pallas as pl
from jax.experimental.pallas import tpu as pltpu
from jax.experimental.pallas import tpu_sc as plsc

N = 10000
D = 128
E = 320000

NC = 2
NS = 16
NW = NC * NS
EPT = E // NW
CHUNK = 100
CPT = 100
SEC = 25

N_SP = 10240
N_DEG = 10240


def _ln_body(x_ref, g_ref, b_ref, o_ref):
    x = x_ref[...]
    mu = jnp.mean(x, axis=1, keepdims=True)
    xc = x - mu
    var = jnp.mean(xc * xc, axis=1, keepdims=True)
    o_ref[...] = xc * lax.rsqrt(var + 1e-5) * g_ref[...] + b_ref[...]


def _layernorm_tc(x, g, b):
    blk = 1000
    return pl.pallas_call(
        _ln_body,
        grid=(N // blk,),
        in_specs=[
            pl.BlockSpec((blk, D), lambda i: (i, 0)),
            pl.BlockSpec((1, D), lambda i: (0, 0)),
            pl.BlockSpec((1, D), lambda i: (0, 0)),
        ],
        out_specs=pl.BlockSpec((blk, D), lambda i: (i, 0)),
        out_shape=jax.ShapeDtypeStruct((N, D), jnp.float32),
    )(x, g.reshape(1, D), b.reshape(1, D))


def _edge_kernel(hn, er, zeros2d, zeros1d,
                 agg_out, deg_out,
                 src_v, dst_v, rowbuf0, rowbuf1, rowbuf2, ones_v,
                 agg_sp, deg_sp, gsem0, gsem1, gsem2, dsem):
    cid = lax.axis_index("c")
    sid = lax.axis_index("s")
    wid = cid * NS + sid

    pltpu.sync_copy(zeros2d, rowbuf0)
    z80 = rowbuf0.at[pl.ds(0, 80)]
    for k in range(8):
        pltpu.sync_copy(z80, agg_sp.at[pl.ds(sid * 640 + k * 80, 80)])
    pltpu.sync_copy(zeros1d.at[pl.ds(sid * 640, 640)],
                    deg_sp.at[pl.ds(sid * 640, 640)])
    for i in range(8):
        ones_v[pl.ds(i * 16, 16)] = jnp.full((16,), 1.0, jnp.float32)
    plsc.subcore_barrier()

    ntrip = (SEC - 1) // 3
    ones_c = ones_v.at[pl.ds(0, CHUNK)]

    def enq(j, buf, sem):
        pltpu.async_copy(hn.at[src_v.at[j]], buf, sem)

    def process(j, buf, sem):
        pltpu.make_async_copy(hn.at[src_v.at[j]], buf, sem).wait()
        pltpu.async_copy(ones_c, deg_sp.at[dst_v.at[j]], dsem, add=True)
        pltpu.sync_copy(buf, agg_sp.at[dst_v.at[j]], add=True)
        pltpu.make_async_copy(ones_c, deg_sp.at[dst_v.at[j]], dsem).wait()

    def body(g, carry):
        a = 3 * g
        process(a, rowbuf0, gsem0)
        enq(a + 3, rowbuf0, gsem0)
        process(a + 1, rowbuf1, gsem1)

        @pl.when(g < ntrip - 1)
        def _():
            enq(a + 4, rowbuf1, gsem1)

        process(a + 2, rowbuf2, gsem2)

        @pl.when(g < ntrip - 1)
        def _():
            enq(a + 5, rowbuf2, gsem2)

        return carry

    for s in range(4):
        pltpu.sync_copy(er.at[wid * 4 + s], src_v)
        pltpu.sync_copy(er.at[NW * 4 + wid * 4 + s], dst_v)
        enq(0, rowbuf0, gsem0)
        enq(1, rowbuf1, gsem1)
        enq(2, rowbuf2, gsem2)
        lax.fori_loop(0, ntrip, body, 0)
        process(SEC - 1, rowbuf0, gsem0)
    plsc.subcore_barrier()

    pltpu.sync_copy(agg_sp.at[pl.ds(sid * 640, 640)],
                    agg_out.at[cid, pl.ds(sid * 640, 640)])
    pltpu.sync_copy(deg_sp.at[pl.ds(sid * 640, 640)],
                    deg_out.at[cid, pl.ds(sid * 640, 640)])


def _edge_aggregate_sc(hn, er, zeros2d, zeros1d):
    mesh = plsc.VectorSubcoreMesh(core_axis_name="c", subcore_axis_name="s")
    return pl.kernel(
        _edge_kernel,
        mesh=mesh,
        out_type=[
            jax.ShapeDtypeStruct((NC, N_SP, D), jnp.float32),
            jax.ShapeDtypeStruct((NC, N_DEG), jnp.float32),
        ],
        scratch_types=[
            pltpu.VMEM((SEC, CHUNK), jnp.int32),
            pltpu.VMEM((SEC, CHUNK), jnp.int32),
            pltpu.VMEM((CHUNK, D), jnp.float32),
            pltpu.VMEM((CHUNK, D), jnp.float32),
            pltpu.VMEM((CHUNK, D), jnp.float32),
            pltpu.VMEM((128,), jnp.float32),
            pltpu.VMEM_SHARED((N_SP, D), jnp.float32),
            pltpu.VMEM_SHARED((N_DEG,), jnp.float32),
            pltpu.SemaphoreType.DMA,
            pltpu.SemaphoreType.DMA,
            pltpu.SemaphoreType.DMA,
            pltpu.SemaphoreType.DMA,
        ],
    )(hn, er, zeros2d, zeros1d)


def _tail_body(hn_ref, a0_ref, a1_ref, d0_ref, d1_ref,
               ws_ref, wn_ref, wsi_ref, b_ref, ing_ref, inb_ref, bsi_ref,
               o_ref):
    hn = hn_ref[...]
    agg = a0_ref[0] + a1_ref[0]
    deg = jnp.maximum(d0_ref[0] + d1_ref[0], 1.0)
    h_neigh = agg / deg
    dn = (((1,), (1,)), ((), ()))
    h_conv = (lax.dot_general(hn, ws_ref[...], dn,
                              preferred_element_type=jnp.float32)
              + lax.dot_general(h_neigh, wn_ref[...], dn,
                                preferred_element_type=jnp.float32)
              + b_ref[...])
    h1 = h_conv + hn
    mu = jnp.mean(h1, axis=1, keepdims=True)
    xc = h1 - mu
    var = jnp.mean(xc * xc, axis=1, keepdims=True)
    h2 = xc * lax.rsqrt(var + 1e-5) * ing_ref[...] + inb_ref[...]
    z = lax.dot_general(h2, wsi_ref[...], dn,
                        preferred_element_type=jnp.float32) + bsi_ref[...]
    h3 = jnp.where(z > 0, z, jnp.exp(jnp.minimum(z, 0.0)) - 1.0)
    o_ref[...] = h3 + h2


def _dense_tail_tc(hn, agg, deg,
                   W_self, W_neigh, W_si, b, in_g, in_b, b_si):
    blk = 1000
    row = lambda i: (i, 0)
    full = lambda i: (0, 0)
    return pl.pallas_call(
        _tail_body,
        grid=(N // blk,),
        in_specs=[
            pl.BlockSpec((blk, D), row),
            pl.BlockSpec((1, blk, D), lambda i: (0, i, 0)),
            pl.BlockSpec((1, blk, D), lambda i: (1, i, 0)),
            pl.BlockSpec((1, blk, 1), lambda i: (0, i, 0)),
            pl.BlockSpec((1, blk, 1), lambda i: (1, i, 0)),
            pl.BlockSpec((D, D), full),
            pl.BlockSpec((D, D), full),
            pl.BlockSpec((D, D), full),
            pl.BlockSpec((1, D), full),
            pl.BlockSpec((1, D), full),
            pl.BlockSpec((1, D), full),
            pl.BlockSpec((1, D), full),
        ],
        out_specs=pl.BlockSpec((blk, D), row),
        out_shape=jax.ShapeDtypeStruct((N, D), jnp.float32),
    )(hn, agg, agg, deg, deg, W_self, W_neigh, W_si,
      b.reshape(1, D), in_g.reshape(1, D), in_b.reshape(1, D),
      b_si.reshape(1, D))


def kernel(h, edge_index, W_self, W_neigh, b, ln_g, ln_b, in_g, in_b,
           W_si, b_si):
    hn = _layernorm_tc(h, ln_g, ln_b)

    er = edge_index.reshape(2 * NW * 4, SEC, CHUNK)
    zeros2d = jnp.zeros((CHUNK, D), jnp.float32)
    zeros1d = jnp.zeros((N_DEG,), jnp.float32)

    agg, deg = _edge_aggregate_sc(hn, er, zeros2d, zeros1d)

    return _dense_tail_tc(hn, agg, deg.reshape(NC, N_DEG, 1),
                          W_self, W_neigh, W_si, b, in_g, in_b, b_si)
